# baseline (speedup 1.0000x reference)
import jax
import jax.numpy as jnp
from jax.experimental import pallas as pl
from jax.experimental.pallas import tpu as pltpu

_B, _T, _D, _NH, _HD = 2, 2048, 1024, 16, 64
_INNER = _NH * _HD
_BT = _B * _T
_CT = 512
_NG = _BT // _CT
_C = 256
_NC = _T // _C
_UNROLL = 2
_NCG = _NC // _UNROLL
_NP = _NH // 2
_ROPE_BASE = 10000.0
_THR_MIN, _THR_MAX = 0.01, 10.0

_f32 = jnp.float32
_bf16 = jnp.bfloat16


def _dot(a, b):
    return jax.lax.dot_general(a, b, (((1,), (0,)), ((), ())),
                               preferred_element_type=_f32)


def _dot_t(a, b):
    return jax.lax.dot_general(a, b, (((1,), (1,)), ((), ())),
                               preferred_element_type=_f32)


def _proj_kernel(al_ref, x_ref, cos_ref, sin_ref, wq_ref, wk_ref,
                 wv_ref, wb_ref, bb_ref,
                 q_ref, k_ref, v_ref, beta_ref, dec_ref, ks_ref, vs_ref,
                 wq_s, wk_s, wv_s):
    g = pl.program_id(0)

    @pl.when(g == 0)
    def _():
        wq_s[...] = wq_ref[...].astype(_bf16)
        wk_s[...] = wk_ref[...].astype(_bf16)
        wv_s[...] = wv_ref[...].astype(_bf16)

    x_hi = x_ref[...].astype(_bf16)

    q_pre = _dot_t(x_hi, wq_s[...])
    beta_ref[...] = jax.nn.sigmoid(
        _dot_t(x_hi, wb_ref[...].astype(_bf16)) + bb_ref[...])
    k_pre = _dot_t(x_hi, wk_s[...])
    v = _dot_t(x_hi, wv_s[...])
    v_ref[...] = v
    vs_ref[0] = jnp.sum(jnp.abs(v), axis=0, keepdims=True)

    c64 = cos_ref[...]
    s64 = sin_ref[...]
    cosf = pltpu.repeat(jnp.concatenate([c64, c64], axis=1), 8, axis=1)
    sinf = pltpu.repeat(jnp.concatenate([s64, s64], axis=1), 8, axis=1)
    lane = jax.lax.broadcasted_iota(jnp.int32, (_CT, _INNER), 1)
    first = (lane & 63) < 32

    def rope(t):
        r1 = jnp.concatenate([t[:, 32:], t[:, :32]], axis=1)
        r2 = jnp.concatenate([t[:, -32:], t[:, :-32]], axis=1)
        return t * cosf + jnp.where(first, r1, r2) * sinf

    q_ref[...] = rope(q_pre).astype(_bf16)
    k_rot = rope(k_pre)
    k_ref[...] = k_rot
    ks_ref[0] = jnp.sum(jnp.abs(k_rot), axis=0, keepdims=True)

    g = pl.program_id(0)
    pi = jax.lax.broadcasted_iota(jnp.int32, (_C, _C), 0)
    ri = jax.lax.broadcasted_iota(jnp.int32, (_C, _C), 1)
    diff = (pi - ri).astype(_f32)
    mask = diff >= 0
    for hh in range(2):
        al = al_ref[2 * g + hh]
        la = jnp.log(jnp.maximum(jax.nn.sigmoid(al), 1e-6))
        dec_ref[hh] = jnp.exp(jnp.where(mask, diff * la, -1e30))


def _attn_kernel(al_ref, q_ref, k_ref, v_ref, b_ref, ks_ref, vs_ref,
                 e_ref, dec_ref, wo_ref, o_ref, s_ref, thr_ref,
                 eq_s, ek_s):
    c = pl.program_id(1)

    @pl.when(c == 0)
    def _():
        s_ref[...] = jnp.zeros_like(s_ref)
        inv_n = 1.0 / (_BT * _INNER)
        thr_ref[0] = jnp.clip(jnp.sum(ks_ref[...]) * inv_n, _THR_MIN, _THR_MAX)
        thr_ref[1] = jnp.clip(jnp.sum(vs_ref[...]) * inv_n, _THR_MIN, _THR_MAX)
        pr = jax.lax.broadcasted_iota(jnp.int32, (_C, _HD), 0).astype(_f32)
        for h in range(_NH):
            lah = jnp.log(jnp.maximum(jax.nn.sigmoid(al_ref[h]), 1e-6))
            eq_s[h] = jnp.exp((pr + 1.0) * lah)
            ek_s[h] = jnp.exp((_C - 1.0 - pr) * lah)

    thr_k = thr_ref[0]
    thr_v = thr_ref[1]
    bfull = _dot(b_ref[...], e_ref[...])
    k2 = k_ref[...]
    kq = jnp.where(k2 > thr_k, 1.0,
                   jnp.where(k2 < -thr_k, -1.0, 0.0)) * bfull
    v2 = v_ref[...]
    vq = jnp.where(v2 > thr_v, 1.0, jnp.where(v2 < -thr_v, -1.0, 0.0))
    q2 = q_ref[...].astype(_f32)

    outs = []
    for hh in range(_NH):
        sl = slice(hh * _HD, (hh + 1) * _HD)
        la = jnp.log(jnp.maximum(jax.nn.sigmoid(al_ref[hh]), 1e-6))
        e_q = eq_s[hh]
        e_k = ek_s[hh]
        a_c = jnp.exp(float(_C) * la)
        dech = dec_ref[hh]

        s = s_ref[hh]
        subouts = []
        for sub in range(_UNROLL):
            rs = slice(sub * _C, (sub + 1) * _C)
            qh = q2[rs, sl]
            kh = kq[rs, sl]
            vh = vq[rs, sl]
            qk = jax.lax.dot_general(qh, kh, (((1,), (1,)), ((), ())),
                                     preferred_element_type=_f32)
            intra = _dot(qk * dech, vh)
            inter = _dot(qh * e_q, s)
            subouts.append(intra + inter)
            kv = jax.lax.dot_general(kh * e_k, vh, (((0,), (0,)), ((), ())),
                                     preferred_element_type=_f32)
            s = s * a_c + kv
        s_ref[hh] = s
        outs.append(jnp.concatenate(subouts, axis=0))

    oh = jnp.concatenate(outs, axis=1).astype(_bf16)
    o_ref[...] = _dot_t(oh, wo_ref[...].astype(_bf16))


def kernel(x, Wq, Wk, Wv, Wo, Wb, bb, alpha_log):
    xf = x.reshape(_BT, _D)

    inv = 1.0 / (_ROPE_BASE ** (jnp.arange(0, _HD, 2, dtype=_f32) / _HD))
    freqs = jnp.arange(_T, dtype=_f32)[:, None] * inv[None, :]
    cos_t = jnp.cos(freqs)
    sin_t = jnp.sin(freqs)
    cos_c = jnp.concatenate([cos_t, cos_t], axis=1)
    sin_c = jnp.concatenate([-sin_t, sin_t], axis=1)

    bbe = bb[None, :]
    al = alpha_log[:, 0]
    ee = (jax.lax.broadcasted_iota(jnp.int32, (_NH, _INNER), 1) // _HD ==
          jax.lax.broadcasted_iota(jnp.int32, (_NH, _INNER), 0)).astype(_f32)

    row_spec = pl.BlockSpec((_CT, _INNER), lambda g: (g, 0))
    w_spec = pl.BlockSpec((_INNER, _D), lambda g: (0, 0))
    smem = pl.BlockSpec(memory_space=pltpu.SMEM)

    q, k, v, beta, dec, ks, vs = pl.pallas_call(
        _proj_kernel,
        grid=(_NG,),
        in_specs=[
            smem,
            pl.BlockSpec((_CT, _D), lambda g: (g, 0)),
            pl.BlockSpec((_CT, _HD), lambda g: (g % (_T // _CT), 0)),
            pl.BlockSpec((_CT, _HD), lambda g: (g % (_T // _CT), 0)),
            w_spec, w_spec, w_spec,
            pl.BlockSpec((_NH, _D), lambda g: (0, 0)),
            pl.BlockSpec((1, _NH), lambda g: (0, 0)),
        ],
        out_specs=[
            row_spec, row_spec, row_spec,
            pl.BlockSpec((_CT, _NH), lambda g: (g, 0)),
            pl.BlockSpec((2, _C, _C), lambda g: (g, 0, 0)),
            pl.BlockSpec((1, 1, _INNER), lambda g: (g, 0, 0)),
            pl.BlockSpec((1, 1, _INNER), lambda g: (g, 0, 0)),
        ],
        out_shape=[
            jax.ShapeDtypeStruct((_BT, _INNER), _bf16),
            jax.ShapeDtypeStruct((_BT, _INNER), _f32),
            jax.ShapeDtypeStruct((_BT, _INNER), _f32),
            jax.ShapeDtypeStruct((_BT, _NH), _f32),
            jax.ShapeDtypeStruct((_NH, _C, _C), _f32),
            jax.ShapeDtypeStruct((_NG, 1, _INNER), _f32),
            jax.ShapeDtypeStruct((_NG, 1, _INNER), _f32),
        ],
        scratch_shapes=[pltpu.VMEM((_INNER, _D), _bf16),
                        pltpu.VMEM((_INNER, _D), _bf16),
                        pltpu.VMEM((_INNER, _D), _bf16)],
        compiler_params=pltpu.CompilerParams(
            dimension_semantics=("arbitrary",)),
    )(al, xf, cos_c, sin_c, Wq, Wk, Wv, Wb, bbe)

    qkvb_spec = pl.BlockSpec(
        (_UNROLL * _C, _INNER), lambda b, c: (b * _NCG + c, 0))
    stat_spec = pl.BlockSpec((_NG, 1, _INNER), lambda b, c: (0, 0, 0))

    out = pl.pallas_call(
        _attn_kernel,
        grid=(_B, _NCG),
        in_specs=[
            smem,
            qkvb_spec, qkvb_spec, qkvb_spec,
            pl.BlockSpec((_UNROLL * _C, _NH), lambda b, c: (b * _NCG + c, 0)),
            stat_spec, stat_spec,
            pl.BlockSpec((_NH, _INNER), lambda b, c: (0, 0)),
            pl.BlockSpec((_NH, _C, _C), lambda b, c: (0, 0, 0)),
            pl.BlockSpec((_D, _INNER), lambda b, c: (0, 0)),
        ],
        out_specs=qkvb_spec,
        out_shape=jax.ShapeDtypeStruct((_BT, _D), _f32),
        scratch_shapes=[pltpu.VMEM((_NH, _HD, _HD), _f32),
                        pltpu.SMEM((2,), _f32),
                        pltpu.VMEM((_NH, _C, _HD), _f32),
                        pltpu.VMEM((_NH, _C, _HD), _f32)],
        compiler_params=pltpu.CompilerParams(
            dimension_semantics=("parallel", "arbitrary")),
    )(al, q, k, v, beta, ks, vs, ee, dec, Wo)

    return out.reshape(_B, _T, _D)


# speedup vs baseline: 1.0977x; 1.0977x over previous
import jax
import jax.numpy as jnp
from jax.experimental import pallas as pl
from jax.experimental.pallas import tpu as pltpu

_B, _T, _D, _NH, _HD = 2, 2048, 1024, 16, 64
_INNER = _NH * _HD
_BT = _B * _T
_CT = 512
_NG = _BT // _CT
_C = 256
_NC = _T // _C
_UNROLL = 4
_NCG = _NC // _UNROLL
_NP = _NH // 2
_ROPE_BASE = 10000.0
_THR_MIN, _THR_MAX = 0.01, 10.0

_f32 = jnp.float32
_bf16 = jnp.bfloat16


def _dot(a, b):
    return jax.lax.dot_general(a, b, (((1,), (0,)), ((), ())),
                               preferred_element_type=_f32)


def _dot_t(a, b):
    return jax.lax.dot_general(a, b, (((1,), (1,)), ((), ())),
                               preferred_element_type=_f32)


def _proj_kernel(al_ref, x_ref, cos_ref, sin_ref, wq_ref, wk_ref,
                 wv_ref, wb_ref, bb_ref,
                 q_ref, k_ref, v_ref, beta_ref, dec_ref, ks_ref, vs_ref):
    x_hi = x_ref[...].astype(_bf16)

    q_pre = _dot_t(x_hi, wq_ref[...].astype(_bf16))
    beta_ref[...] = jax.nn.sigmoid(
        _dot_t(x_hi, wb_ref[...].astype(_bf16)) + bb_ref[...])
    k_pre = _dot_t(x_hi, wk_ref[...].astype(_bf16))
    v = _dot_t(x_hi, wv_ref[...].astype(_bf16))
    v_ref[...] = v
    vs_ref[0] = jnp.sum(jnp.abs(v), axis=0, keepdims=True)

    c64 = cos_ref[...]
    s64 = sin_ref[...]
    cosf = pltpu.repeat(jnp.concatenate([c64, c64], axis=1), 8, axis=1)
    sinf = pltpu.repeat(jnp.concatenate([s64, s64], axis=1), 8, axis=1)
    lane = jax.lax.broadcasted_iota(jnp.int32, (_CT, _INNER), 1)
    first = (lane & 63) < 32

    def rope(t):
        r1 = jnp.concatenate([t[:, 32:], t[:, :32]], axis=1)
        r2 = jnp.concatenate([t[:, -32:], t[:, :-32]], axis=1)
        return t * cosf + jnp.where(first, r1, r2) * sinf

    q_ref[...] = rope(q_pre).astype(_bf16)
    k_rot = rope(k_pre)
    k_ref[...] = k_rot
    ks_ref[0] = jnp.sum(jnp.abs(k_rot), axis=0, keepdims=True)

    g = pl.program_id(0)
    pi = jax.lax.broadcasted_iota(jnp.int32, (_C, _C), 0)
    ri = jax.lax.broadcasted_iota(jnp.int32, (_C, _C), 1)
    diff = (pi - ri).astype(_f32)
    mask = diff >= 0
    for hh in range(2):
        al = al_ref[2 * g + hh]
        la = jnp.log(jnp.maximum(jax.nn.sigmoid(al), 1e-6))
        dec_ref[hh] = jnp.exp(jnp.where(mask, diff * la, -1e30))


def _attn_kernel(al_ref, q_ref, k_ref, v_ref, b_ref, ks_ref, vs_ref,
                 e_ref, dec_ref, wo_ref, o_ref, s_ref, thr_ref):
    c = pl.program_id(1)

    @pl.when(c == 0)
    def _():
        s_ref[...] = jnp.zeros_like(s_ref)
        inv_n = 1.0 / (_BT * _INNER)
        thr_ref[0] = jnp.clip(jnp.sum(ks_ref[...]) * inv_n, _THR_MIN, _THR_MAX)
        thr_ref[1] = jnp.clip(jnp.sum(vs_ref[...]) * inv_n, _THR_MIN, _THR_MAX)

    thr_k = thr_ref[0]
    thr_v = thr_ref[1]
    bfull = _dot(b_ref[...], e_ref[...])
    k2 = k_ref[...]
    kq = jnp.where(k2 > thr_k, 1.0,
                   jnp.where(k2 < -thr_k, -1.0, 0.0)) * bfull
    v2 = v_ref[...]
    vq = jnp.where(v2 > thr_v, 1.0, jnp.where(v2 < -thr_v, -1.0, 0.0))
    q2 = q_ref[...].astype(_f32)

    prow = jax.lax.broadcasted_iota(jnp.int32, (_C, _HD), 0).astype(_f32)
    outs = []
    for hh in range(_NH):
        sl = slice(hh * _HD, (hh + 1) * _HD)
        la = jnp.log(jnp.maximum(jax.nn.sigmoid(al_ref[hh]), 1e-6))
        e_q = jnp.exp((prow + 1.0) * la)
        e_k = jnp.exp((_C - 1.0 - prow) * la)
        a_c = jnp.exp(float(_C) * la)
        dech = dec_ref[hh]

        s = s_ref[hh]
        subouts = []
        for sub in range(_UNROLL):
            rs = slice(sub * _C, (sub + 1) * _C)
            qh = q2[rs, sl]
            kh = kq[rs, sl]
            vh = vq[rs, sl]
            qk = jax.lax.dot_general(qh, kh, (((1,), (1,)), ((), ())),
                                     preferred_element_type=_f32)
            intra = _dot(qk * dech, vh)
            inter = _dot(qh * e_q, s)
            subouts.append(intra + inter)
            kv = jax.lax.dot_general(kh * e_k, vh, (((0,), (0,)), ((), ())),
                                     preferred_element_type=_f32)
            s = s * a_c + kv
        s_ref[hh] = s
        outs.append(jnp.concatenate(subouts, axis=0))

    oh = jnp.concatenate(outs, axis=1).astype(_bf16)
    o_ref[...] = _dot_t(oh, wo_ref[...].astype(_bf16))


def kernel(x, Wq, Wk, Wv, Wo, Wb, bb, alpha_log):
    xf = x.reshape(_BT, _D)

    inv = 1.0 / (_ROPE_BASE ** (jnp.arange(0, _HD, 2, dtype=_f32) / _HD))
    freqs = jnp.arange(_T, dtype=_f32)[:, None] * inv[None, :]
    cos_t = jnp.cos(freqs)
    sin_t = jnp.sin(freqs)
    cos_c = jnp.concatenate([cos_t, cos_t], axis=1)
    sin_c = jnp.concatenate([-sin_t, sin_t], axis=1)

    bbe = bb[None, :]
    al = alpha_log[:, 0]
    ee = (jax.lax.broadcasted_iota(jnp.int32, (_NH, _INNER), 1) // _HD ==
          jax.lax.broadcasted_iota(jnp.int32, (_NH, _INNER), 0)).astype(_f32)

    row_spec = pl.BlockSpec((_CT, _INNER), lambda g: (g, 0))
    w_spec = pl.BlockSpec((_INNER, _D), lambda g: (0, 0))
    smem = pl.BlockSpec(memory_space=pltpu.SMEM)

    q, k, v, beta, dec, ks, vs = pl.pallas_call(
        _proj_kernel,
        grid=(_NG,),
        in_specs=[
            smem,
            pl.BlockSpec((_CT, _D), lambda g: (g, 0)),
            pl.BlockSpec((_CT, _HD), lambda g: (g % (_T // _CT), 0)),
            pl.BlockSpec((_CT, _HD), lambda g: (g % (_T // _CT), 0)),
            w_spec, w_spec, w_spec,
            pl.BlockSpec((_NH, _D), lambda g: (0, 0)),
            pl.BlockSpec((1, _NH), lambda g: (0, 0)),
        ],
        out_specs=[
            row_spec, row_spec, row_spec,
            pl.BlockSpec((_CT, _NH), lambda g: (g, 0)),
            pl.BlockSpec((2, _C, _C), lambda g: (g, 0, 0)),
            pl.BlockSpec((1, 1, _INNER), lambda g: (g, 0, 0)),
            pl.BlockSpec((1, 1, _INNER), lambda g: (g, 0, 0)),
        ],
        out_shape=[
            jax.ShapeDtypeStruct((_BT, _INNER), _bf16),
            jax.ShapeDtypeStruct((_BT, _INNER), _f32),
            jax.ShapeDtypeStruct((_BT, _INNER), _f32),
            jax.ShapeDtypeStruct((_BT, _NH), _f32),
            jax.ShapeDtypeStruct((_NH, _C, _C), _f32),
            jax.ShapeDtypeStruct((_NG, 1, _INNER), _f32),
            jax.ShapeDtypeStruct((_NG, 1, _INNER), _f32),
        ],
        compiler_params=pltpu.CompilerParams(
            dimension_semantics=("parallel",)),
    )(al, xf, cos_c, sin_c, Wq, Wk, Wv, Wb, bbe)

    qkvb_spec = pl.BlockSpec(
        (_UNROLL * _C, _INNER), lambda b, c: (b * _NCG + c, 0))
    stat_spec = pl.BlockSpec((_NG, 1, _INNER), lambda b, c: (0, 0, 0))

    out = pl.pallas_call(
        _attn_kernel,
        grid=(_B, _NCG),
        in_specs=[
            smem,
            qkvb_spec, qkvb_spec, qkvb_spec,
            pl.BlockSpec((_UNROLL * _C, _NH), lambda b, c: (b * _NCG + c, 0)),
            stat_spec, stat_spec,
            pl.BlockSpec((_NH, _INNER), lambda b, c: (0, 0)),
            pl.BlockSpec((_NH, _C, _C), lambda b, c: (0, 0, 0)),
            pl.BlockSpec((_D, _INNER), lambda b, c: (0, 0)),
        ],
        out_specs=qkvb_spec,
        out_shape=jax.ShapeDtypeStruct((_BT, _D), _f32),
        scratch_shapes=[pltpu.VMEM((_NH, _HD, _HD), _f32),
                        pltpu.SMEM((2,), _f32)],
        compiler_params=pltpu.CompilerParams(
            dimension_semantics=("parallel", "arbitrary")),
    )(al, q, k, v, beta, ks, vs, ee, dec, Wo)

    return out.reshape(_B, _T, _D)


# speedup vs baseline: 1.0988x; 1.0010x over previous
import jax
import jax.numpy as jnp
from jax.experimental import pallas as pl
from jax.experimental.pallas import tpu as pltpu

_B, _T, _D, _NH, _HD = 2, 2048, 1024, 16, 64
_INNER = _NH * _HD
_BT = _B * _T
_CT = 512
_NG = _BT // _CT
_C = 256
_NC = _T // _C
_UNROLL = 4
_NCG = _NC // _UNROLL
_ROPE_BASE = 10000.0
_THR_MIN, _THR_MAX = 0.01, 10.0

_f32 = jnp.float32
_bf16 = jnp.bfloat16


def _dot(a, b):
    return jax.lax.dot_general(a, b, (((1,), (0,)), ((), ())),
                               preferred_element_type=_f32)


def _dot_t(a, b):
    return jax.lax.dot_general(a, b, (((1,), (1,)), ((), ())),
                               preferred_element_type=_f32)


def _proj_kernel(al_ref, x_ref, cos_ref, sin_ref, wq_ref, wk_ref,
                 wv_ref, wb_ref, bb_ref,
                 q_ref, k_ref, v_ref, beta_ref, dec_ref, ks_ref, vs_ref):
    x_hi = x_ref[...].astype(_bf16)

    q_pre = _dot_t(x_hi, wq_ref[...].astype(_bf16))
    beta_ref[...] = jax.nn.sigmoid(
        _dot_t(x_hi, wb_ref[...].astype(_bf16)) + bb_ref[...])
    k_pre = _dot_t(x_hi, wk_ref[...].astype(_bf16))
    v = _dot_t(x_hi, wv_ref[...].astype(_bf16))
    v_ref[...] = v
    vs_ref[0] = jnp.sum(jnp.abs(v), axis=0, keepdims=True)

    c64 = cos_ref[...]
    s64 = sin_ref[...]
    cosf = pltpu.repeat(jnp.concatenate([c64, c64], axis=1), 8, axis=1)
    sinf = pltpu.repeat(jnp.concatenate([s64, s64], axis=1), 8, axis=1)
    lane = jax.lax.broadcasted_iota(jnp.int32, (_CT, _INNER), 1)
    first = (lane & 63) < 32

    def rope(t):
        r1 = jnp.concatenate([t[:, 32:], t[:, :32]], axis=1)
        r2 = jnp.concatenate([t[:, -32:], t[:, :-32]], axis=1)
        return t * cosf + jnp.where(first, r1, r2) * sinf

    q_ref[...] = rope(q_pre).astype(_bf16)
    k_rot = rope(k_pre)
    k_ref[...] = k_rot
    ks_ref[0] = jnp.sum(jnp.abs(k_rot), axis=0, keepdims=True)

    g = pl.program_id(0)
    pi = jax.lax.broadcasted_iota(jnp.int32, (_C, _C), 0)
    ri = jax.lax.broadcasted_iota(jnp.int32, (_C, _C), 1)
    diff = (pi - ri).astype(_f32)
    mask = diff >= 0
    for hh in range(2):
        al = al_ref[2 * g + hh]
        la = jnp.log(jnp.maximum(jax.nn.sigmoid(al), 1e-6))
        dec_ref[hh] = jnp.exp(jnp.where(mask, diff * la, -1e30))


def _attn_kernel(al_ref, q_ref, k_ref, v_ref, b_ref, ks_ref, vs_ref,
                 e_ref, dec_ref, wo_ref, o_ref, s_ref, thr_ref):
    c = pl.program_id(1)

    @pl.when(c == 0)
    def _():
        s_ref[...] = jnp.zeros_like(s_ref)
        inv_n = 1.0 / (_BT * _INNER)
        thr_ref[0] = jnp.clip(jnp.sum(ks_ref[...]) * inv_n, _THR_MIN, _THR_MAX)
        thr_ref[1] = jnp.clip(jnp.sum(vs_ref[...]) * inv_n, _THR_MIN, _THR_MAX)

    thr_k = thr_ref[0]
    thr_v = thr_ref[1]
    bfull = _dot(b_ref[...], e_ref[...])
    k2 = k_ref[...]
    kq = jnp.where(k2 > thr_k, 1.0,
                   jnp.where(k2 < -thr_k, -1.0, 0.0)) * bfull
    v2 = v_ref[...]
    vq = jnp.where(v2 > thr_v, 1.0, jnp.where(v2 < -thr_v, -1.0, 0.0))
    q2 = q_ref[...].astype(_f32)

    prow = jax.lax.broadcasted_iota(jnp.int32, (_C, _HD), 0).astype(_f32)
    outs = []
    for hh in range(_NH):
        sl = slice(hh * _HD, (hh + 1) * _HD)
        la = jnp.log(jnp.maximum(jax.nn.sigmoid(al_ref[hh]), 1e-6))
        e_q = jnp.exp((prow + 1.0) * la)
        e_k = jnp.exp((_C - 1.0 - prow) * la)
        a_c = jnp.exp(float(_C) * la)
        dech = dec_ref[hh]

        s = s_ref[hh]
        subouts = []
        for sub in range(_UNROLL):
            rs = slice(sub * _C, (sub + 1) * _C)
            qh = q2[rs, sl]
            kh = kq[rs, sl]
            vh = vq[rs, sl]
            qk = jax.lax.dot_general(qh, kh, (((1,), (1,)), ((), ())),
                                     preferred_element_type=_f32)
            intra = _dot(qk * dech, vh)
            inter = _dot(qh * e_q, s)
            subouts.append(intra + inter)
            kv = jax.lax.dot_general(kh * e_k, vh, (((0,), (0,)), ((), ())),
                                     preferred_element_type=_f32)
            s = s * a_c + kv
        s_ref[hh] = s
        outs.append(jnp.concatenate(subouts, axis=0))

    oh = jnp.concatenate(outs, axis=1).astype(_bf16)
    o_ref[...] = _dot_t(oh, wo_ref[...].astype(_bf16))


def kernel(x, Wq, Wk, Wv, Wo, Wb, bb, alpha_log):
    xf = x.reshape(_BT, _D)

    inv = 1.0 / (_ROPE_BASE ** (jnp.arange(0, _HD, 2, dtype=_f32) / _HD))
    freqs = jnp.arange(_T, dtype=_f32)[:, None] * inv[None, :]
    cos_t = jnp.cos(freqs)
    sin_t = jnp.sin(freqs)
    cos_c = jnp.concatenate([cos_t, cos_t], axis=1)
    sin_c = jnp.concatenate([-sin_t, sin_t], axis=1)

    bbe = bb[None, :]
    al = alpha_log[:, 0]
    ee = (jax.lax.broadcasted_iota(jnp.int32, (_NH, _INNER), 1) // _HD ==
          jax.lax.broadcasted_iota(jnp.int32, (_NH, _INNER), 0)).astype(_f32)

    row_spec = pl.BlockSpec((_CT, _INNER), lambda g: (g, 0))
    w_spec = pl.BlockSpec((_INNER, _D), lambda g: (0, 0))
    smem = pl.BlockSpec(memory_space=pltpu.SMEM)

    q, k, v, beta, dec, ks, vs = pl.pallas_call(
        _proj_kernel,
        grid=(_NG,),
        in_specs=[
            smem,
            pl.BlockSpec((_CT, _D), lambda g: (g, 0)),
            pl.BlockSpec((_CT, _HD), lambda g: (g % (_T // _CT), 0)),
            pl.BlockSpec((_CT, _HD), lambda g: (g % (_T // _CT), 0)),
            w_spec, w_spec, w_spec,
            pl.BlockSpec((_NH, _D), lambda g: (0, 0)),
            pl.BlockSpec((1, _NH), lambda g: (0, 0)),
        ],
        out_specs=[
            row_spec, row_spec, row_spec,
            pl.BlockSpec((_CT, _NH), lambda g: (g, 0)),
            pl.BlockSpec((2, _C, _C), lambda g: (g, 0, 0)),
            pl.BlockSpec((1, 1, _INNER), lambda g: (g, 0, 0)),
            pl.BlockSpec((1, 1, _INNER), lambda g: (g, 0, 0)),
        ],
        out_shape=[
            jax.ShapeDtypeStruct((_BT, _INNER), _bf16),
            jax.ShapeDtypeStruct((_BT, _INNER), _f32),
            jax.ShapeDtypeStruct((_BT, _INNER), _f32),
            jax.ShapeDtypeStruct((_BT, _NH), _f32),
            jax.ShapeDtypeStruct((_NH, _C, _C), _f32),
            jax.ShapeDtypeStruct((_NG, 1, _INNER), _f32),
            jax.ShapeDtypeStruct((_NG, 1, _INNER), _f32),
        ],
        compiler_params=pltpu.CompilerParams(
            dimension_semantics=("parallel",)),
    )(al, xf, cos_c, sin_c, Wq, Wk, Wv, Wb, bbe)

    qkvb_spec = pl.BlockSpec(
        (_UNROLL * _C, _INNER), lambda b, c: (b * _NCG + c, 0))
    stat_spec = pl.BlockSpec((_NG, 1, _INNER), lambda b, c: (0, 0, 0))

    out = pl.pallas_call(
        _attn_kernel,
        grid=(_B, _NCG),
        in_specs=[
            smem,
            qkvb_spec, qkvb_spec, qkvb_spec,
            pl.BlockSpec((_UNROLL * _C, _NH), lambda b, c: (b * _NCG + c, 0)),
            stat_spec, stat_spec,
            pl.BlockSpec((_NH, _INNER), lambda b, c: (0, 0)),
            pl.BlockSpec((_NH, _C, _C), lambda b, c: (0, 0, 0)),
            pl.BlockSpec((_D, _INNER), lambda b, c: (0, 0)),
        ],
        out_specs=qkvb_spec,
        out_shape=jax.ShapeDtypeStruct((_BT, _D), _f32),
        scratch_shapes=[pltpu.VMEM((_NH, _HD, _HD), _f32),
                        pltpu.SMEM((2,), _f32)],
        compiler_params=pltpu.CompilerParams(
            dimension_semantics=("parallel", "arbitrary")),
    )(al, q, k, v, beta, ks, vs, ee, dec, Wo)

    return out.reshape(_B, _T, _D)
